# speedup vs baseline: 1.0325x; 1.0325x over previous
import jax
import jax.numpy as jnp
from jax.experimental import pallas as pl


def _matmul_kernel(x_ref, w_ref, o_ref):
    o_ref[...] = jnp.dot(x_ref[...], w_ref[...],
                         preferred_element_type=jnp.float32)


def _matmul(x, w):
    n, d = x.shape
    d2 = w.shape[1]
    return pl.pallas_call(
        _matmul_kernel,
        out_shape=jax.ShapeDtypeStruct((n, d2), jnp.float32),
        grid=(n // 2000,),
        in_specs=[pl.BlockSpec((2000, d), lambda i: (i, 0)),
                  pl.BlockSpec((d, d2), lambda i: (0, 0))],
        out_specs=pl.BlockSpec((2000, d2), lambda i: (i, 0)),
    )(x, w)


def kernel(x, edge_index, WU, WV, WA, WB, Wout):
    N, D = x.shape
    L = WU.shape[0]
    src = edge_index[0]
    dst = edge_index[1]
    h = x
    for l in range(L):
        Ux = _matmul(h, WU[l])
        Vx = _matmul(h, WV[l])
        Ax = _matmul(h, WA[l])
        Bx = _matmul(h, WB[l])
        e = Ax[dst] + Bx[src]
        sig = jax.nn.sigmoid(e)
        num = jax.ops.segment_sum(sig * Vx[src], dst, num_segments=N)
        den = jax.ops.segment_sum(sig, dst, num_segments=N)
        out = Ux + num / (den + 1e-6)
        h = h + jax.nn.relu(out)
    return _matmul(h, Wout)


# speedup vs baseline: 2.9942x; 2.9000x over previous
import functools

import jax
import jax.numpy as jnp
from jax import lax
from jax.experimental import pallas as pl
from jax.experimental.pallas import tpu as pltpu
from jax.experimental.pallas import tpu_sc as plsc

N = 10000
D = 128
E = 320000
NW = 32
BROWS = 320
NPAD = NW * BROWS
K = 64
K8 = K + 8
EPAD = E + 2 * K8
NV = D // 16


def _proj_block(p, ux_ref, ax_ref, bv_ref):
    ux_ref[...] = p[:, 0:D]
    ax_ref[...] = p[:, D:2 * D]
    bv_ref[:, 0, :] = p[:, 2 * D:3 * D]
    bv_ref[:, 1, :] = p[:, 3 * D:4 * D]


def _proj0_body(x_ref, w_ref, ux_ref, ax_ref, bv_ref):
    p = jnp.dot(x_ref[...], w_ref[...], preferred_element_type=jnp.float32)
    _proj_block(p, ux_ref, ax_ref, bv_ref)


def _cproj_body(h_ref, uxp_ref, r_ref, w_ref, h_out, ux_ref, ax_ref, bv_ref):
    h = h_ref[...] + jnp.maximum(uxp_ref[...] + r_ref[...], 0.0)
    h_out[...] = h
    p = jnp.dot(h, w_ref[...], preferred_element_type=jnp.float32)
    _proj_block(p, ux_ref, ax_ref, bv_ref)


def _final_body(h_ref, uxp_ref, r_ref, w_ref, o_ref):
    h = h_ref[...] + jnp.maximum(uxp_ref[...] + r_ref[...], 0.0)
    o_ref[...] = jnp.dot(h, w_ref[...], preferred_element_type=jnp.float32)


_BN = 2000


def _proj0(x, wcat):
    return pl.pallas_call(
        _proj0_body,
        grid=(N // _BN,),
        in_specs=[pl.BlockSpec((_BN, D), lambda i: (i, 0)),
                  pl.BlockSpec((D, 4 * D), lambda i: (0, 0))],
        out_specs=[pl.BlockSpec((_BN, D), lambda i: (i, 0)),
                   pl.BlockSpec((_BN, D), lambda i: (i, 0)),
                   pl.BlockSpec((_BN, 2, D), lambda i: (i, 0, 0))],
        out_shape=[jax.ShapeDtypeStruct((N, D), jnp.float32),
                   jax.ShapeDtypeStruct((N, D), jnp.float32),
                   jax.ShapeDtypeStruct((N, 2, D), jnp.float32)],
    )(x, wcat)


def _cproj(h, uxp, r, wcat):
    return pl.pallas_call(
        _cproj_body,
        grid=(N // _BN,),
        in_specs=[pl.BlockSpec((_BN, D), lambda i: (i, 0)),
                  pl.BlockSpec((_BN, D), lambda i: (i, 0)),
                  pl.BlockSpec((_BN, D), lambda i: (i, 0)),
                  pl.BlockSpec((D, 4 * D), lambda i: (0, 0))],
        out_specs=[pl.BlockSpec((_BN, D), lambda i: (i, 0)),
                   pl.BlockSpec((_BN, D), lambda i: (i, 0)),
                   pl.BlockSpec((_BN, D), lambda i: (i, 0)),
                   pl.BlockSpec((_BN, 2, D), lambda i: (i, 0, 0))],
        out_shape=[jax.ShapeDtypeStruct((N, D), jnp.float32),
                   jax.ShapeDtypeStruct((N, D), jnp.float32),
                   jax.ShapeDtypeStruct((N, D), jnp.float32),
                   jax.ShapeDtypeStruct((N, 2, D), jnp.float32)],
    )(h, uxp, r, wcat)


def _final(h, uxp, r, wout):
    return pl.pallas_call(
        _final_body,
        grid=(N // _BN,),
        in_specs=[pl.BlockSpec((_BN, D), lambda i: (i, 0)),
                  pl.BlockSpec((_BN, D), lambda i: (i, 0)),
                  pl.BlockSpec((_BN, D), lambda i: (i, 0)),
                  pl.BlockSpec((D, D), lambda i: (0, 0))],
        out_specs=pl.BlockSpec((_BN, D), lambda i: (i, 0)),
        out_shape=jax.ShapeDtypeStruct((N, D), jnp.float32),
    )(h, uxp, r, wout)


def _edge_body(dst_h, src_h, bnd_h, ax_h, bv_h, r_h,
               bnd_v, idxd_v, idxs_v, a_v, bv_v, blk_v, sem_a, sem_b):
    c = lax.axis_index("c")
    s = lax.axis_index("s")
    wid = s * 2 + c
    n_lo = wid * BROWS

    zv = jnp.zeros((16,), jnp.float32)

    def zero_row(i, _):
        for j in range(NV):
            blk_v[pl.ds(i * D + 16 * j, 16)] = zv
        return 0

    lax.fori_loop(0, BROWS, zero_row, 0)

    pltpu.sync_copy(bnd_h, bnd_v)
    bvec = bnd_v[pl.ds(wid, 16)]
    e0 = bvec[0]
    e1 = bvec[1]
    nchunks = (e1 - e0 + (K - 1)) // K

    def flush(cur, num, den):
        r = cur - n_lo
        for j in range(NV):
            blk_v[pl.ds(r * D + 16 * j, 16)] = num[j] / (den[j] + 1e-6)

    def chunk_body(ci, carry):
        base = e0 + ci * K
        abase = (base // 8) * 8
        off = base - abase
        cnt = jnp.minimum(K, e1 - base)
        pltpu.sync_copy(dst_h.at[pl.ds(abase, K8)], idxd_v.at[pl.ds(0, K8)])
        pltpu.sync_copy(src_h.at[pl.ds(abase, K8)], idxs_v)
        cp_a = pltpu.async_copy(ax_h.at[idxd_v.at[pl.ds(0, K8)]], a_v, sem_a)
        cp_b = pltpu.async_copy(bv_h.at[idxs_v], bv_v, sem_b)
        cp_a.wait()
        cp_b.wait()

        def edge_body(i, ec):
            cur = ec[0]
            num = ec[1:1 + NV]
            den = ec[1 + NV:1 + 2 * NV]
            d = idxd_v[pl.ds(i, 16)][0]
            is_new = d != cur

            @pl.when(jnp.logical_and(is_new, cur >= 0))
            def _():
                flush(cur, num, den)

            num2 = []
            den2 = []
            for j in range(NV):
                a = a_v[i, pl.ds(16 * j, 16)]
                b = bv_v[i, 0, pl.ds(16 * j, 16)]
                v = bv_v[i, 1, pl.ds(16 * j, 16)]
                sig = 1.0 / (1.0 + jnp.exp(-(a + b)))
                num2.append(jnp.where(is_new, zv, num[j]) + sig * v)
                den2.append(jnp.where(is_new, zv, den[j]) + sig)
            return (d,) + tuple(num2) + tuple(den2)

        return lax.fori_loop(off, off + cnt, edge_body, carry)

    init = (jnp.int32(-1),) + (zv,) * (2 * NV)
    fc = lax.fori_loop(0, nchunks, chunk_body, init)
    cur = fc[0]
    num = fc[1:1 + NV]
    den = fc[1 + NV:1 + 2 * NV]

    @pl.when(cur >= 0)
    def _():
        flush(cur, num, den)

    pltpu.sync_copy(blk_v, r_h.at[pl.ds(n_lo * D, BROWS * D)])


_edge_kernel = pl.kernel(
    _edge_body,
    out_type=jax.ShapeDtypeStruct((NPAD * D,), jnp.float32),
    mesh=plsc.VectorSubcoreMesh(core_axis_name="c", subcore_axis_name="s"),
    scratch_types=[
        pltpu.VMEM((48,), jnp.int32),
        pltpu.VMEM((K8 + 16,), jnp.int32),
        pltpu.VMEM((K8,), jnp.int32),
        pltpu.VMEM((K8, D), jnp.float32),
        pltpu.VMEM((K8, 2, D), jnp.float32),
        pltpu.VMEM((BROWS * D,), jnp.float32),
        pltpu.SemaphoreType.DMA,
        pltpu.SemaphoreType.DMA,
    ],
)


def kernel(x, edge_index, WU, WV, WA, WB, Wout):
    L = WU.shape[0]
    src = edge_index[0]
    dst = edge_index[1]
    dst_s, src_s = lax.sort([dst, src], num_keys=1)
    pad = jnp.zeros((EPAD - E,), jnp.int32)
    dst_p = jnp.concatenate([dst_s, pad])
    src_p = jnp.concatenate([src_s, pad])
    node_lo = jnp.arange(33, dtype=jnp.int32) * BROWS
    bounds = jnp.searchsorted(dst_s, node_lo, side="left").astype(jnp.int32)
    bounds = jnp.concatenate([bounds, jnp.full((15,), E, jnp.int32)])

    wcats = [jnp.concatenate([WU[l], WA[l], WB[l], WV[l]], axis=1)
             for l in range(L)]

    ux, ax, bv = _proj0(x, wcats[0])
    r = _edge_kernel(dst_p, src_p, bounds, ax, bv).reshape(NPAD, D)[:N]
    h = x
    for l in range(1, L):
        h, ux, ax, bv = _cproj(h, ux, r, wcats[l])
        r = _edge_kernel(dst_p, src_p, bounds, ax, bv).reshape(NPAD, D)[:N]
    return _final(h, ux, r, Wout)


# speedup vs baseline: 5.0331x; 1.6809x over previous
import functools

import jax
import jax.numpy as jnp
from jax import lax
from jax.experimental import pallas as pl
from jax.experimental.pallas import tpu as pltpu
from jax.experimental.pallas import tpu_sc as plsc

N = 10000
D = 128
E = 320000
NW = 32
BROWS = 320
NPAD = NW * BROWS
K = 64
K8 = K + 8
EPAD = E + 2 * K8
NV = D // 16


def _proj_block(p, ux_ref, ax_ref, bv_ref):
    ux_ref[...] = p[:, 0:D]
    ax_ref[...] = p[:, D:2 * D]
    bv_ref[:, 0, :] = p[:, 2 * D:3 * D]
    bv_ref[:, 1, :] = p[:, 3 * D:4 * D]


def _proj0_body(x_ref, w_ref, ux_ref, ax_ref, bv_ref):
    p = jnp.dot(x_ref[...], w_ref[...], preferred_element_type=jnp.float32)
    _proj_block(p, ux_ref, ax_ref, bv_ref)


def _cproj_body(h_ref, uxp_ref, r_ref, w_ref, h_out, ux_ref, ax_ref, bv_ref):
    h = h_ref[...] + jnp.maximum(uxp_ref[...] + r_ref[...], 0.0)
    h_out[...] = h
    p = jnp.dot(h, w_ref[...], preferred_element_type=jnp.float32)
    _proj_block(p, ux_ref, ax_ref, bv_ref)


def _final_body(h_ref, uxp_ref, r_ref, w_ref, o_ref):
    h = h_ref[...] + jnp.maximum(uxp_ref[...] + r_ref[...], 0.0)
    o_ref[...] = jnp.dot(h, w_ref[...], preferred_element_type=jnp.float32)


_BN = 2000


def _proj0(x, wcat):
    return pl.pallas_call(
        _proj0_body,
        grid=(N // _BN,),
        in_specs=[pl.BlockSpec((_BN, D), lambda i: (i, 0)),
                  pl.BlockSpec((D, 4 * D), lambda i: (0, 0))],
        out_specs=[pl.BlockSpec((_BN, D), lambda i: (i, 0)),
                   pl.BlockSpec((_BN, D), lambda i: (i, 0)),
                   pl.BlockSpec((_BN, 2, D), lambda i: (i, 0, 0))],
        out_shape=[jax.ShapeDtypeStruct((N, D), jnp.float32),
                   jax.ShapeDtypeStruct((N, D), jnp.float32),
                   jax.ShapeDtypeStruct((N, 2, D), jnp.float32)],
    )(x, wcat)


def _cproj(h, uxp, r, wcat):
    return pl.pallas_call(
        _cproj_body,
        grid=(N // _BN,),
        in_specs=[pl.BlockSpec((_BN, D), lambda i: (i, 0)),
                  pl.BlockSpec((_BN, D), lambda i: (i, 0)),
                  pl.BlockSpec((_BN, D), lambda i: (i, 0)),
                  pl.BlockSpec((D, 4 * D), lambda i: (0, 0))],
        out_specs=[pl.BlockSpec((_BN, D), lambda i: (i, 0)),
                   pl.BlockSpec((_BN, D), lambda i: (i, 0)),
                   pl.BlockSpec((_BN, D), lambda i: (i, 0)),
                   pl.BlockSpec((_BN, 2, D), lambda i: (i, 0, 0))],
        out_shape=[jax.ShapeDtypeStruct((N, D), jnp.float32),
                   jax.ShapeDtypeStruct((N, D), jnp.float32),
                   jax.ShapeDtypeStruct((N, D), jnp.float32),
                   jax.ShapeDtypeStruct((N, 2, D), jnp.float32)],
    )(h, uxp, r, wcat)


def _final(h, uxp, r, wout):
    return pl.pallas_call(
        _final_body,
        grid=(N // _BN,),
        in_specs=[pl.BlockSpec((_BN, D), lambda i: (i, 0)),
                  pl.BlockSpec((_BN, D), lambda i: (i, 0)),
                  pl.BlockSpec((_BN, D), lambda i: (i, 0)),
                  pl.BlockSpec((D, D), lambda i: (0, 0))],
        out_specs=pl.BlockSpec((_BN, D), lambda i: (i, 0)),
        out_shape=jax.ShapeDtypeStruct((N, D), jnp.float32),
    )(h, uxp, r, wout)


def _edge_body(dst_h, src_h, bnd_h, ax_h, bv_h, r_h,
               bnd_v, idxd0, idxd1, idxs0, idxs1, a0, a1, b0, b1, blk_v,
               sid0, sid1, sis0, sis1, sa0, sa1, sb0, sb1):
    idxd = (idxd0, idxd1)
    idxs = (idxs0, idxs1)
    av = (a0, a1)
    bvv = (b0, b1)
    sid = (sid0, sid1)
    sis = (sis0, sis1)
    sa = (sa0, sa1)
    sb = (sb0, sb1)

    c = lax.axis_index("c")
    s = lax.axis_index("s")
    wid = s * 2 + c
    n_lo = wid * BROWS

    zv = jnp.zeros((16,), jnp.float32)

    def zero_row(i, _):
        for j in range(NV):
            blk_v[pl.ds(i * D + 16 * j, 16)] = zv
        return 0

    lax.fori_loop(0, BROWS, zero_row, 0)

    pltpu.sync_copy(bnd_h, bnd_v)
    bvec = bnd_v[pl.ds(wid, 16)]
    e0 = bvec[0]
    e1 = bvec[1]
    nchunks = (e1 - e0 + (K - 1)) // K

    def flush(cur, num, den):
        r = cur - n_lo
        for j in range(NV):
            blk_v[pl.ds(r * D + 16 * j, 16)] = num[j] / (den[j] + 1e-6)

    def abase_of(ci):
        return ((e0 + ci * K) // 8) * 8

    def issue_idx(ci, sl):
        ab = abase_of(ci)
        pltpu.async_copy(dst_h.at[pl.ds(ab, K8)],
                         idxd[sl].at[pl.ds(0, K8)], sid[sl])
        pltpu.async_copy(src_h.at[pl.ds(ab, K8)], idxs[sl], sis[sl])

    def wait_idx(ci, sl):
        ab = abase_of(ci)
        pltpu.make_async_copy(dst_h.at[pl.ds(ab, K8)],
                              idxd[sl].at[pl.ds(0, K8)], sid[sl]).wait()
        pltpu.make_async_copy(src_h.at[pl.ds(ab, K8)], idxs[sl],
                              sis[sl]).wait()

    def issue_gather(sl):
        pltpu.async_copy(ax_h.at[idxd[sl].at[pl.ds(0, K8)]], av[sl], sa[sl])
        pltpu.async_copy(bv_h.at[idxs[sl]], bvv[sl], sb[sl])

    def wait_gather(sl):
        pltpu.make_async_copy(ax_h.at[idxd[sl].at[pl.ds(0, K8)]], av[sl],
                              sa[sl]).wait()
        pltpu.make_async_copy(bv_h.at[idxs[sl]], bvv[sl], sb[sl]).wait()

    @pl.when(nchunks > 0)
    def _():
        issue_idx(0, 0)
        wait_idx(0, 0)
        issue_gather(0)

    @pl.when(nchunks > 1)
    def _():
        issue_idx(1, 1)

    def compute_chunk(ci, sl, carry, a_v, bv_v, idxd_v):
        base = e0 + ci * K
        off = base - abase_of(ci)
        cnt = jnp.minimum(K, e1 - base)

        def edge_body(i, ec):
            cur = ec[0]
            num = ec[1:1 + NV]
            den = ec[1 + NV:1 + 2 * NV]
            d = idxd_v[pl.ds(i, 16)][0]
            is_new = d != cur

            @pl.when(jnp.logical_and(is_new, cur >= 0))
            def _():
                flush(cur, num, den)

            num2 = []
            den2 = []
            for j in range(NV):
                a = a_v[i, pl.ds(16 * j, 16)]
                b = bv_v[i, 0, pl.ds(16 * j, 16)]
                v = bv_v[i, 1, pl.ds(16 * j, 16)]
                sig = 1.0 / (1.0 + jnp.exp(-(a + b)))
                num2.append(jnp.where(is_new, zv, num[j]) + sig * v)
                den2.append(jnp.where(is_new, zv, den[j]) + sig)
            return (d,) + tuple(num2) + tuple(den2)

        return lax.fori_loop(off, off + cnt, edge_body, carry)

    def pair_body(p, carry):
        for b in range(2):
            ci = 2 * p + b
            s0 = b
            s1 = 1 - b

            @pl.when(ci < nchunks)
            def _():
                wait_gather(s0)

            @pl.when(ci + 1 < nchunks)
            def _():
                wait_idx(ci + 1, s1)
                issue_gather(s1)

            carry = compute_chunk(ci, s0, carry, av[s0], bvv[s0], idxd[s0])

            @pl.when(ci + 2 < nchunks)
            def _():
                issue_idx(ci + 2, s0)
        return carry

    npairs = (nchunks + 1) // 2
    init = (jnp.int32(-1),) + (zv,) * (2 * NV)
    fc = lax.fori_loop(0, npairs, pair_body, init)
    cur = fc[0]
    num = fc[1:1 + NV]
    den = fc[1 + NV:1 + 2 * NV]

    @pl.when(cur >= 0)
    def _():
        flush(cur, num, den)

    pltpu.sync_copy(blk_v, r_h.at[pl.ds(n_lo * D, BROWS * D)])


_edge_kernel = pl.kernel(
    _edge_body,
    out_type=jax.ShapeDtypeStruct((NPAD * D,), jnp.float32),
    mesh=plsc.VectorSubcoreMesh(core_axis_name="c", subcore_axis_name="s"),
    scratch_types=[
        pltpu.VMEM((48,), jnp.int32),
        pltpu.VMEM((K8 + 16,), jnp.int32),
        pltpu.VMEM((K8 + 16,), jnp.int32),
        pltpu.VMEM((K8,), jnp.int32),
        pltpu.VMEM((K8,), jnp.int32),
        pltpu.VMEM((K8, D), jnp.float32),
        pltpu.VMEM((K8, D), jnp.float32),
        pltpu.VMEM((K8, 2, D), jnp.float32),
        pltpu.VMEM((K8, 2, D), jnp.float32),
        pltpu.VMEM((BROWS * D,), jnp.float32),
    ] + [pltpu.SemaphoreType.DMA] * 8,
)


def kernel(x, edge_index, WU, WV, WA, WB, Wout):
    L = WU.shape[0]
    src = edge_index[0]
    dst = edge_index[1]
    dst_s, src_s = lax.sort([dst, src], num_keys=1)
    pad = jnp.zeros((EPAD - E,), jnp.int32)
    dst_p = jnp.concatenate([dst_s, pad])
    src_p = jnp.concatenate([src_s, pad])
    node_lo = jnp.arange(33, dtype=jnp.int32) * BROWS
    bounds = jnp.searchsorted(dst_s, node_lo, side="left").astype(jnp.int32)
    bounds = jnp.concatenate([bounds, jnp.full((15,), E, jnp.int32)])

    wcats = [jnp.concatenate([WU[l], WA[l], WB[l], WV[l]], axis=1)
             for l in range(L)]

    ux, ax, bv = _proj0(x, wcats[0])
    r = _edge_kernel(dst_p, src_p, bounds, ax, bv).reshape(NPAD, D)[:N]
    h = x
    for l in range(1, L):
        h, ux, ax, bv = _cproj(h, ux, r, wcats[l])
        r = _edge_kernel(dst_p, src_p, bounds, ax, bv).reshape(NPAD, D)[:N]
    return _final(h, ux, r, Wout)


# speedup vs baseline: 5.8944x; 1.1711x over previous
import functools

import jax
import jax.numpy as jnp
from jax import lax
from jax.experimental import pallas as pl
from jax.experimental.pallas import tpu as pltpu
from jax.experimental.pallas import tpu_sc as plsc

N = 10000
D = 128
E = 320000
NW = 32
BROWS = 320
NPAD = NW * BROWS
K = 64
K8 = K + 8
EPAD = E + 2 * K8
NV = D // 16


def _proj_block(p, ux_ref, ax_ref, bv_ref):
    ux_ref[...] = p[:, 0:D]
    ax_ref[...] = p[:, D:2 * D]
    bv_ref[:, 0, :] = p[:, 2 * D:3 * D]
    bv_ref[:, 1, :] = p[:, 3 * D:4 * D]


def _proj0_body(x_ref, w_ref, ux_ref, ax_ref, bv_ref):
    p = jnp.dot(x_ref[...], w_ref[...], preferred_element_type=jnp.float32)
    _proj_block(p, ux_ref, ax_ref, bv_ref)


def _cproj_body(h_ref, uxp_ref, r_ref, w_ref, h_out, ux_ref, ax_ref, bv_ref):
    h = h_ref[...] + jnp.maximum(uxp_ref[...] + r_ref[...], 0.0)
    h_out[...] = h
    p = jnp.dot(h, w_ref[...], preferred_element_type=jnp.float32)
    _proj_block(p, ux_ref, ax_ref, bv_ref)


def _final_body(h_ref, uxp_ref, r_ref, w_ref, o_ref):
    h = h_ref[...] + jnp.maximum(uxp_ref[...] + r_ref[...], 0.0)
    o_ref[...] = jnp.dot(h, w_ref[...], preferred_element_type=jnp.float32)


_BN = 2000


def _proj0(x, wcat):
    return pl.pallas_call(
        _proj0_body,
        grid=(N // _BN,),
        in_specs=[pl.BlockSpec((_BN, D), lambda i: (i, 0)),
                  pl.BlockSpec((D, 4 * D), lambda i: (0, 0))],
        out_specs=[pl.BlockSpec((_BN, D), lambda i: (i, 0)),
                   pl.BlockSpec((_BN, D), lambda i: (i, 0)),
                   pl.BlockSpec((_BN, 2, D), lambda i: (i, 0, 0))],
        out_shape=[jax.ShapeDtypeStruct((N, D), jnp.float32),
                   jax.ShapeDtypeStruct((N, D), jnp.float32),
                   jax.ShapeDtypeStruct((N, 2, D), jnp.float32)],
    )(x, wcat)


def _cproj(h, uxp, r, wcat):
    return pl.pallas_call(
        _cproj_body,
        grid=(N // _BN,),
        in_specs=[pl.BlockSpec((_BN, D), lambda i: (i, 0)),
                  pl.BlockSpec((_BN, D), lambda i: (i, 0)),
                  pl.BlockSpec((_BN, D), lambda i: (i, 0)),
                  pl.BlockSpec((D, 4 * D), lambda i: (0, 0))],
        out_specs=[pl.BlockSpec((_BN, D), lambda i: (i, 0)),
                   pl.BlockSpec((_BN, D), lambda i: (i, 0)),
                   pl.BlockSpec((_BN, D), lambda i: (i, 0)),
                   pl.BlockSpec((_BN, 2, D), lambda i: (i, 0, 0))],
        out_shape=[jax.ShapeDtypeStruct((N, D), jnp.float32),
                   jax.ShapeDtypeStruct((N, D), jnp.float32),
                   jax.ShapeDtypeStruct((N, D), jnp.float32),
                   jax.ShapeDtypeStruct((N, 2, D), jnp.float32)],
    )(h, uxp, r, wcat)


def _final(h, uxp, r, wout):
    return pl.pallas_call(
        _final_body,
        grid=(N // _BN,),
        in_specs=[pl.BlockSpec((_BN, D), lambda i: (i, 0)),
                  pl.BlockSpec((_BN, D), lambda i: (i, 0)),
                  pl.BlockSpec((_BN, D), lambda i: (i, 0)),
                  pl.BlockSpec((D, D), lambda i: (0, 0))],
        out_specs=pl.BlockSpec((_BN, D), lambda i: (i, 0)),
        out_shape=jax.ShapeDtypeStruct((N, D), jnp.float32),
    )(h, uxp, r, wout)


def _edge_body(dst_h, src_h, bnd_h, ax_h, bv_h, r_h,
               bnd_v, idxd0, idxd1, idxs0, idxs1, a_blk, b0, b1, blk_v,
               sid0, sid1, sis0, sis1, sb0, sb1):
    idxd = (idxd0, idxd1)
    idxs = (idxs0, idxs1)
    bvv = (b0, b1)
    sid = (sid0, sid1)
    sis = (sis0, sis1)
    sb = (sb0, sb1)

    c = lax.axis_index("c")
    s = lax.axis_index("s")
    wid = s * 2 + c
    n_lo = wid * BROWS

    zv = jnp.zeros((16,), jnp.float32)

    def zero_row(i, _):
        for j in range(NV):
            blk_v[pl.ds(i * D + 16 * j, 16)] = zv
        return 0

    lax.fori_loop(0, BROWS, zero_row, 0)

    pltpu.sync_copy(ax_h.at[pl.ds(n_lo, BROWS)], a_blk)
    pltpu.sync_copy(bnd_h, bnd_v)
    bvec = bnd_v[pl.ds(wid, 16)]
    e0 = bvec[0]
    e1 = bvec[1]
    nchunks = (e1 - e0 + (K - 1)) // K

    def flush(cur, num, den):
        r = cur - n_lo
        for j in range(NV):
            blk_v[pl.ds(r * D + 16 * j, 16)] = num[j] / (den[j] + 1e-6)

    def abase_of(ci):
        return ((e0 + ci * K) // 8) * 8

    def issue_idx(ci, sl):
        ab = abase_of(ci)
        pltpu.async_copy(dst_h.at[pl.ds(ab, K8)],
                         idxd[sl].at[pl.ds(0, K8)], sid[sl])
        pltpu.async_copy(src_h.at[pl.ds(ab, K8)], idxs[sl], sis[sl])

    def wait_idx(ci, sl):
        ab = abase_of(ci)
        pltpu.make_async_copy(dst_h.at[pl.ds(ab, K8)],
                              idxd[sl].at[pl.ds(0, K8)], sid[sl]).wait()
        pltpu.make_async_copy(src_h.at[pl.ds(ab, K8)], idxs[sl],
                              sis[sl]).wait()

    def issue_gather(sl):
        pltpu.async_copy(bv_h.at[idxs[sl]], bvv[sl], sb[sl])

    def wait_gather(sl):
        pltpu.make_async_copy(bv_h.at[idxs[sl]], bvv[sl], sb[sl]).wait()

    @pl.when(nchunks > 0)
    def _():
        issue_idx(0, 0)
        wait_idx(0, 0)
        issue_gather(0)

    @pl.when(nchunks > 1)
    def _():
        issue_idx(1, 1)

    def compute_chunk(ci, carry, bv_v, idxd_v):
        base = e0 + ci * K
        off = base - abase_of(ci)
        cnt = jnp.minimum(K, e1 - base)

        def edge_body(i, ec):
            cur = ec[0]
            num = ec[1:1 + NV]
            den = ec[1 + NV:1 + 2 * NV]
            d = idxd_v[pl.ds(i, 16)][0]
            is_new = d != cur
            r = d - n_lo

            @pl.when(jnp.logical_and(is_new, cur >= 0))
            def _():
                flush(cur, num, den)

            num2 = []
            den2 = []
            for j in range(NV):
                a = a_blk[r, pl.ds(16 * j, 16)]
                b = bv_v[i, 0, pl.ds(16 * j, 16)]
                v = bv_v[i, 1, pl.ds(16 * j, 16)]
                sig = 1.0 / (1.0 + jnp.exp(-(a + b)))
                num2.append(jnp.where(is_new, zv, num[j]) + sig * v)
                den2.append(jnp.where(is_new, zv, den[j]) + sig)
            return (d,) + tuple(num2) + tuple(den2)

        return lax.fori_loop(off, off + cnt, edge_body, carry)

    def pair_body(p, carry):
        for b in range(2):
            ci = 2 * p + b
            s0 = b
            s1 = 1 - b

            @pl.when(ci < nchunks)
            def _():
                wait_gather(s0)

            @pl.when(ci + 1 < nchunks)
            def _():
                wait_idx(ci + 1, s1)
                issue_gather(s1)

            carry = compute_chunk(ci, carry, bvv[s0], idxd[s0])

            @pl.when(ci + 2 < nchunks)
            def _():
                issue_idx(ci + 2, s0)
        return carry

    npairs = (nchunks + 1) // 2
    init = (jnp.int32(-1),) + (zv,) * (2 * NV)
    fc = lax.fori_loop(0, npairs, pair_body, init)
    cur = fc[0]
    num = fc[1:1 + NV]
    den = fc[1 + NV:1 + 2 * NV]

    @pl.when(cur >= 0)
    def _():
        flush(cur, num, den)

    pltpu.sync_copy(blk_v, r_h.at[pl.ds(n_lo * D, BROWS * D)])


_edge_kernel = pl.kernel(
    _edge_body,
    out_type=jax.ShapeDtypeStruct((NPAD * D,), jnp.float32),
    mesh=plsc.VectorSubcoreMesh(core_axis_name="c", subcore_axis_name="s"),
    scratch_types=[
        pltpu.VMEM((48,), jnp.int32),
        pltpu.VMEM((K8 + 16,), jnp.int32),
        pltpu.VMEM((K8 + 16,), jnp.int32),
        pltpu.VMEM((K8,), jnp.int32),
        pltpu.VMEM((K8,), jnp.int32),
        pltpu.VMEM((BROWS, D), jnp.float32),
        pltpu.VMEM((K8, 2, D), jnp.float32),
        pltpu.VMEM((K8, 2, D), jnp.float32),
        pltpu.VMEM((BROWS * D,), jnp.float32),
    ] + [pltpu.SemaphoreType.DMA] * 6,
)


def kernel(x, edge_index, WU, WV, WA, WB, Wout):
    L = WU.shape[0]
    src = edge_index[0]
    dst = edge_index[1]
    dst_s, src_s = lax.sort([dst, src], num_keys=1)
    pad = jnp.zeros((EPAD - E,), jnp.int32)
    dst_p = jnp.concatenate([dst_s, pad])
    src_p = jnp.concatenate([src_s, pad])
    node_lo = jnp.arange(33, dtype=jnp.int32) * BROWS
    bounds = jnp.searchsorted(dst_s, node_lo, side="left").astype(jnp.int32)
    bounds = jnp.concatenate([bounds, jnp.full((15,), E, jnp.int32)])

    wcats = [jnp.concatenate([WU[l], WA[l], WB[l], WV[l]], axis=1)
             for l in range(L)]

    ux, ax, bv = _proj0(x, wcats[0])
    r = _edge_kernel(dst_p, src_p, bounds, ax, bv).reshape(NPAD, D)[:N]
    h = x
    for l in range(1, L):
        h, ux, ax, bv = _cproj(h, ux, r, wcats[l])
        r = _edge_kernel(dst_p, src_p, bounds, ax, bv).reshape(NPAD, D)[:N]
    return _final(h, ux, r, Wout)
